# baseline (speedup 1.0000x reference)
import functools

import numpy as np

import jax
import jax.numpy as jnp
from jax import lax
from jax.experimental import pallas as pl
from jax.experimental.pallas import tpu as pltpu
from jax.experimental.pallas import tpu_sc as plsc

B = 16384
NUM_NUM = 13
F = 26
VOCAB = 100000
EMB = 32
HID = 128
OUT = 64
EPS = 1e-5

NC = 2
NS = 16
NW = NC * NS
FS = 28
TOT = B * FS
PER_W = TOT // NW
IDX_ROWS = PER_W // 128
CHUNK_ROWS = 8
NCHUNK = IDX_ROWS // CHUNK_ROWS
CHUNK = CHUNK_ROWS * 128


def _sc_gather(tables_flat, idxT):
    mesh = plsc.VectorSubcoreMesh(core_axis_name="c", subcore_axis_name="s")
    BW = B // NW

    @functools.partial(
        pl.kernel,
        mesh=mesh,
        compiler_params=pltpu.CompilerParams(use_tc_tiling_on_sc=False,
                                             needs_layout_passes=False),
        out_type=jax.ShapeDtypeStruct((TOT, EMB), jnp.float32),
        scratch_types=[
            pltpu.VMEM((FS, BW), jnp.int32),
            pltpu.VMEM((PER_W,), jnp.int32),
            pltpu.VMEM((CHUNK, EMB), jnp.float32),
            pltpu.SemaphoreType.DMA,
        ],
    )
    def k(tab_hbm, idxT_hbm, out_hbm, idxm, idxp, rows_v, sem):
        wid = lax.axis_index("s") * NC + lax.axis_index("c")
        pltpu.sync_copy(idxT_hbm.at[:, pl.ds(wid * BW, BW)], idxm)

        def octet_body(o, carry):
            for ti in range(FS * 8 // 16):
                t = lax.iota(jnp.int32, 16) + (16 * ti)
                q = t >> 5
                r = (t & 31) >> 2
                j = t & 3
                vals = plsc.load_gather(idxm, [4 * q + j, (o << 3) + r])
                idxp[pl.ds(o * (FS * 8) + 16 * ti, 16)] = vals
            return carry

        lax.fori_loop(0, BW // 8, octet_body, 0)

        def chunk_body(c, carry):
            copies = []
            for j in range(CHUNK_ROWS):
                copies.append(
                    pltpu.async_copy(
                        tab_hbm.at[idxp.at[pl.ds((c * CHUNK_ROWS + j) * 128,
                                                 128)]],
                        rows_v.at[pl.ds(j * 128, 128)],
                        sem,
                    )
                )
            for cp in copies:
                cp.wait()
            out_base = wid * PER_W + c * CHUNK
            pltpu.sync_copy(rows_v, out_hbm.at[pl.ds(out_base, CHUNK)])
            return carry

        lax.fori_loop(0, NCHUNK, chunk_body, 0)

    return k(tables_flat, idxT)


SB = 196
VROWS = SB * 128


RL_KS = 14
RL_STEPS = SB // RL_KS


def _relayout_body(in_ref, e_ref, out_ref):
    a = in_ref[0]
    e_sel = e_ref[...]
    for s in range(RL_KS):
        base = 512 * s
        if s % 4 == 0:
            out_ref[pl.ds(128 * s, 128), :] = jnp.concatenate(
                [a[:, base:base + 128].T,
                 a[:, base + 128:base + 256].T,
                 a[:, base + 256:base + 384].T,
                 a[:, base + 384:base + 512].T], axis=1)
        else:
            ab = a[:, base:base + 512].astype(jnp.bfloat16)
            mx = jax.lax.dot_general(
                ab[:, 0:128], e_sel[0],
                (((0,), (0,)), ((), ())), preferred_element_type=jnp.float32)
            for j in range(1, 4):
                mx = mx + jax.lax.dot_general(
                    ab[:, 128 * j:128 * (j + 1)], e_sel[j],
                    (((0,), (0,)), ((), ())),
                    preferred_element_type=jnp.float32)
            out_ref[pl.ds(128 * s, 128), :] = mx


def _relayout(tables_t):
    e_host = np.zeros((4, 32, 128), np.float32)
    for j in range(4):
        e_host[j, np.arange(32), 32 * j + np.arange(32)] = 1.0
    e_np = jnp.asarray(e_host, dtype=jnp.bfloat16)
    return pl.pallas_call(
        _relayout_body,
        grid=(F, RL_STEPS),
        compiler_params=pltpu.CompilerParams(
            fuse_transposed_lhs_in_matmul=True),
        in_specs=[
            pl.BlockSpec((1, EMB, 512 * RL_KS), lambda f, s: (f, 0, s)),
            pl.BlockSpec((4, EMB, 128), lambda f, s: (0, 0, 0)),
        ],
        out_specs=pl.BlockSpec((128 * RL_KS, 128),
                               lambda f, s: (f * RL_STEPS + s, 0)),
        out_shape=jax.ShapeDtypeStruct((F * VROWS, 128), jnp.float32),
    )(tables_t, e_np)


BT = 1024
T = B // BT


def _mlp1_body(xn_ref, xc_ref, w1n_ref, w1c_ref, b1_ref, h_ref, stats_ref):
    i = pl.program_id(0)
    h = jnp.dot(xn_ref[...], w1n_ref[...], preferred_element_type=jnp.float32)
    for q in range(FS // 4):
        xq = xc_ref[:, q].reshape(BT, 128)
        h = h + jnp.dot(xq, w1c_ref[q], preferred_element_type=jnp.float32)
    h = jnp.maximum(h + b1_ref[...], 0.0)
    h_ref[...] = h

    @pl.when(i == 0)
    def _():
        stats_ref[...] = jnp.zeros_like(stats_ref)

    stats_ref[0:1, :] += jnp.sum(h, axis=0, keepdims=True)
    stats_ref[1:2, :] += jnp.sum(h * h, axis=0, keepdims=True)


def _mlp1(xn, xc, W1n, W1c, b1):
    return pl.pallas_call(
        _mlp1_body,
        grid=(T,),
        in_specs=[
            pl.BlockSpec((BT, NUM_NUM), lambda i: (i, 0)),
            pl.BlockSpec((BT // 8, FS // 4, 8, 128), lambda i: (i, 0, 0, 0)),
            pl.BlockSpec((NUM_NUM, HID), lambda i: (0, 0)),
            pl.BlockSpec((FS // 4, 128, HID), lambda i: (0, 0, 0)),
            pl.BlockSpec((1, HID), lambda i: (0, 0)),
        ],
        out_specs=[
            pl.BlockSpec((BT, HID), lambda i: (i, 0)),
            pl.BlockSpec((2, HID), lambda i: (0, 0)),
        ],
        out_shape=[
            jax.ShapeDtypeStruct((B, HID), jnp.float32),
            jax.ShapeDtypeStruct((2, HID), jnp.float32),
        ],
    )(xn, xc, W1n, W1c, b1)


def _mlp2_body(h_ref, stats_ref, g_ref, be_ref, w2_ref, b2_ref, out_ref):
    stats = stats_ref[...]
    mean = stats[0:1, :] * (1.0 / B)
    var = stats[1:2, :] * (1.0 / B) - mean * mean
    inv = lax.rsqrt(var + EPS)
    scale = g_ref[...] * inv
    shift = be_ref[...] - mean * scale
    hn = h_ref[...] * scale + shift
    out_ref[...] = (
        lax.dot_general(w2_ref[...], hn, (((0,), (1,)), ((), ())),
                        preferred_element_type=jnp.float32)
        + b2_ref[...]
    )


def _mlp2(h, stats, gamma, beta, W2, b2):
    return pl.pallas_call(
        _mlp2_body,
        grid=(T,),
        in_specs=[
            pl.BlockSpec((BT, HID), lambda i: (i, 0)),
            pl.BlockSpec((2, HID), lambda i: (0, 0)),
            pl.BlockSpec((1, HID), lambda i: (0, 0)),
            pl.BlockSpec((1, HID), lambda i: (0, 0)),
            pl.BlockSpec((HID, OUT), lambda i: (0, 0)),
            pl.BlockSpec((OUT, 1), lambda i: (0, 0)),
        ],
        out_specs=pl.BlockSpec((OUT, BT), lambda i: (0, i)),
        out_shape=jax.ShapeDtypeStruct((OUT, B), jnp.float32),
    )(h, stats, gamma, beta, W2, b2)


def kernel(numerical_x, categorical_x, tables, W1, b1, gamma, beta, W2, b2):
    tables_flat = _relayout(tables.transpose(0, 2, 1)).reshape(F * VROWS * 4, EMB)
    cat = categorical_x.T
    idxT = (4 * ((cat >> 9) * 128 + (cat & 127)) + ((cat >> 7) & 3)
            + (jnp.arange(F, dtype=jnp.int32) * (4 * VROWS))[:, None])
    dummy = jnp.broadcast_to(idxT[0:1, :] & 0x7FFC, (FS - F, B))
    idxT_full = jnp.concatenate([idxT, dummy], axis=0)
    xcat = _sc_gather(tables_flat, idxT_full).reshape(B // 8, FS // 4, 8, 128)

    W1n = W1[:NUM_NUM]
    W1c = jnp.concatenate(
        [W1[NUM_NUM:], jnp.zeros(((FS - F) * EMB, HID), jnp.float32)],
        axis=0).reshape(FS // 4, 128, HID)
    h, stats = _mlp1(numerical_x, xcat, W1n, W1c, b1.reshape(1, HID))
    out_t = _mlp2(h, stats, gamma.reshape(1, HID), beta.reshape(1, HID), W2,
                  b2.reshape(OUT, 1))
    return out_t.T


# speedup vs baseline: 1.3179x; 1.3179x over previous
import functools

import numpy as np

import jax
import jax.numpy as jnp
from jax import lax
from jax.experimental import pallas as pl
from jax.experimental.pallas import tpu as pltpu
from jax.experimental.pallas import tpu_sc as plsc

B = 16384
NUM_NUM = 13
F = 26
VOCAB = 100000
EMB = 32
HID = 128
OUT = 64
EPS = 1e-5

NC = 2
NS = 16
NW = NC * NS
FS = 28
TOT = B * FS
PER_W = TOT // NW
IDX_ROWS = PER_W // 128
CHUNK_ROWS = 8
NCHUNK = IDX_ROWS // CHUNK_ROWS
CHUNK = CHUNK_ROWS * 128


def _sc_gather(tables_flat, idxT):
    mesh = plsc.VectorSubcoreMesh(core_axis_name="c", subcore_axis_name="s")
    BW = B // NW

    @functools.partial(
        pl.kernel,
        mesh=mesh,
        compiler_params=pltpu.CompilerParams(use_tc_tiling_on_sc=False,
                                             needs_layout_passes=False),
        out_type=jax.ShapeDtypeStruct((TOT, EMB), jnp.float32),
        scratch_types=[
            pltpu.VMEM((FS, BW), jnp.int32),
            pltpu.VMEM((PER_W,), jnp.int32),
            pltpu.VMEM((CHUNK, EMB), jnp.float32),
            pltpu.SemaphoreType.DMA,
        ],
    )
    def k(tab_hbm, idxT_hbm, out_hbm, idxm, idxp, rows_v, sem):
        wid = lax.axis_index("s") * NC + lax.axis_index("c")
        pltpu.sync_copy(idxT_hbm.at[:, pl.ds(wid * BW, BW)], idxm)

        def octet_body(o, carry):
            for ti in range(FS * 8 // 16):
                t = lax.iota(jnp.int32, 16) + (16 * ti)
                q = t >> 5
                r = (t & 31) >> 2
                j = t & 3
                vals = plsc.load_gather(idxm, [4 * q + j, (o << 3) + r])
                idxp[pl.ds(o * (FS * 8) + 16 * ti, 16)] = vals
            return carry

        lax.fori_loop(0, BW // 8, octet_body, 0)

        def chunk_body(c, carry):
            copies = []
            for j in range(CHUNK_ROWS):
                copies.append(
                    pltpu.async_copy(
                        tab_hbm.at[idxp.at[pl.ds((c * CHUNK_ROWS + j) * 128,
                                                 128)]],
                        rows_v.at[pl.ds(j * 128, 128)],
                        sem,
                    )
                )
            for cp in copies:
                cp.wait()
            out_base = wid * PER_W + c * CHUNK
            pltpu.sync_copy(rows_v, out_hbm.at[pl.ds(out_base, CHUNK)])
            return carry

        lax.fori_loop(0, NCHUNK, chunk_body, 0)

    return k(tables_flat, idxT)


SB = 196
VROWS = SB * 128


RL_KS = 49
RL_STEPS = SB // RL_KS


def _relayout_body(in_ref, e_ref, out_ref):
    a = in_ref[0]
    e_sel = e_ref[...]
    for s in range(RL_KS):
        base = 512 * s
        if s % 4 == 0:
            out_ref[pl.ds(128 * s, 128), :] = jnp.concatenate(
                [a[:, base:base + 128].T,
                 a[:, base + 128:base + 256].T,
                 a[:, base + 256:base + 384].T,
                 a[:, base + 384:base + 512].T], axis=1)
        else:
            ab = a[:, base:base + 512].astype(jnp.bfloat16)
            mx = jax.lax.dot_general(
                ab[:, 0:128], e_sel[0],
                (((0,), (0,)), ((), ())), preferred_element_type=jnp.float32)
            for j in range(1, 4):
                mx = mx + jax.lax.dot_general(
                    ab[:, 128 * j:128 * (j + 1)], e_sel[j],
                    (((0,), (0,)), ((), ())),
                    preferred_element_type=jnp.float32)
            out_ref[pl.ds(128 * s, 128), :] = mx


def _relayout(tables_t):
    e_host = np.zeros((4, 32, 128), np.float32)
    for j in range(4):
        e_host[j, np.arange(32), 32 * j + np.arange(32)] = 1.0
    e_np = jnp.asarray(e_host, dtype=jnp.bfloat16)
    return pl.pallas_call(
        _relayout_body,
        grid=(F, RL_STEPS),
        compiler_params=pltpu.CompilerParams(
            fuse_transposed_lhs_in_matmul=True),
        in_specs=[
            pl.BlockSpec((1, EMB, 512 * RL_KS), lambda f, s: (f, 0, s)),
            pl.BlockSpec((4, EMB, 128), lambda f, s: (0, 0, 0)),
        ],
        out_specs=pl.BlockSpec((128 * RL_KS, 128),
                               lambda f, s: (f * RL_STEPS + s, 0)),
        out_shape=jax.ShapeDtypeStruct((F * VROWS, 128), jnp.float32),
    )(tables_t, e_np)


BT = 1024
T = B // BT


def _mlp1_body(xn_ref, xc_ref, w1n_ref, w1c_ref, b1_ref, h_ref, stats_ref):
    i = pl.program_id(0)
    h = jnp.dot(xn_ref[...], w1n_ref[...], preferred_element_type=jnp.float32)
    for q in range(FS // 4):
        xq = xc_ref[:, q].reshape(BT, 128)
        h = h + jnp.dot(xq, w1c_ref[q], preferred_element_type=jnp.float32)
    h = jnp.maximum(h + b1_ref[...], 0.0)
    h_ref[...] = h

    @pl.when(i == 0)
    def _():
        stats_ref[...] = jnp.zeros_like(stats_ref)

    stats_ref[0:1, :] += jnp.sum(h, axis=0, keepdims=True)
    stats_ref[1:2, :] += jnp.sum(h * h, axis=0, keepdims=True)


def _mlp1(xn, xc, W1n, W1c, b1):
    return pl.pallas_call(
        _mlp1_body,
        grid=(T,),
        in_specs=[
            pl.BlockSpec((BT, NUM_NUM), lambda i: (i, 0)),
            pl.BlockSpec((BT // 8, FS // 4, 8, 128), lambda i: (i, 0, 0, 0)),
            pl.BlockSpec((NUM_NUM, HID), lambda i: (0, 0)),
            pl.BlockSpec((FS // 4, 128, HID), lambda i: (0, 0, 0)),
            pl.BlockSpec((1, HID), lambda i: (0, 0)),
        ],
        out_specs=[
            pl.BlockSpec((BT, HID), lambda i: (i, 0)),
            pl.BlockSpec((2, HID), lambda i: (0, 0)),
        ],
        out_shape=[
            jax.ShapeDtypeStruct((B, HID), jnp.float32),
            jax.ShapeDtypeStruct((2, HID), jnp.float32),
        ],
    )(xn, xc, W1n, W1c, b1)


def _mlp2_body(h_ref, stats_ref, g_ref, be_ref, w2_ref, b2_ref, out_ref):
    stats = stats_ref[...]
    mean = stats[0:1, :] * (1.0 / B)
    var = stats[1:2, :] * (1.0 / B) - mean * mean
    inv = lax.rsqrt(var + EPS)
    scale = g_ref[...] * inv
    shift = be_ref[...] - mean * scale
    hn = h_ref[...] * scale + shift
    out_ref[...] = (
        lax.dot_general(w2_ref[...], hn, (((0,), (1,)), ((), ())),
                        preferred_element_type=jnp.float32)
        + b2_ref[...]
    )


def _mlp2(h, stats, gamma, beta, W2, b2):
    return pl.pallas_call(
        _mlp2_body,
        grid=(T,),
        in_specs=[
            pl.BlockSpec((BT, HID), lambda i: (i, 0)),
            pl.BlockSpec((2, HID), lambda i: (0, 0)),
            pl.BlockSpec((1, HID), lambda i: (0, 0)),
            pl.BlockSpec((1, HID), lambda i: (0, 0)),
            pl.BlockSpec((HID, OUT), lambda i: (0, 0)),
            pl.BlockSpec((OUT, 1), lambda i: (0, 0)),
        ],
        out_specs=pl.BlockSpec((OUT, BT), lambda i: (0, i)),
        out_shape=jax.ShapeDtypeStruct((OUT, B), jnp.float32),
    )(h, stats, gamma, beta, W2, b2)


def kernel(numerical_x, categorical_x, tables, W1, b1, gamma, beta, W2, b2):
    tables_flat = _relayout(tables.transpose(0, 2, 1)).reshape(F * VROWS * 4, EMB)
    cat = categorical_x.T
    idxT = (4 * ((cat >> 9) * 128 + (cat & 127)) + ((cat >> 7) & 3)
            + (jnp.arange(F, dtype=jnp.int32) * (4 * VROWS))[:, None])
    dummy = jnp.broadcast_to(idxT[0:1, :] & 0x7FFC, (FS - F, B))
    idxT_full = jnp.concatenate([idxT, dummy], axis=0)
    xcat = _sc_gather(tables_flat, idxT_full).reshape(B // 8, FS // 4, 8, 128)

    W1n = W1[:NUM_NUM]
    W1c = jnp.concatenate(
        [W1[NUM_NUM:], jnp.zeros(((FS - F) * EMB, HID), jnp.float32)],
        axis=0).reshape(FS // 4, 128, HID)
    h, stats = _mlp1(numerical_x, xcat, W1n, W1c, b1.reshape(1, HID))
    out_t = _mlp2(h, stats, gamma.reshape(1, HID), beta.reshape(1, HID), W2,
                  b2.reshape(OUT, 1))
    return out_t.T
